# baseline (speedup 1.0000x reference)
import functools

import jax
import jax.numpy as jnp
from jax import lax
from jax.experimental import pallas as pl
from jax.experimental.pallas import tpu as pltpu
from jax.experimental.pallas import tpu_sc as plsc

D_MODEL = 1024
T_TOTAL = 8192
B_BATCH = 4
PE_ROWS = 50
PE_PAD = 64
OFFSET = 20

NUM_CORES = 2
NUM_SUBCORES = 16
NW = NUM_CORES * NUM_SUBCORES

T_SC = 3072
TS_PER_W = T_SC // NW
CH = 4
NCHUNK = TS_PER_W // CH
IDX_ROW = 128
NSLOT = 4
NGRP = NCHUNK // NSLOT
LANES = 16
DCH = D_MODEL // LANES

TBLK = 256
N_SC_BLK = T_SC // TBLK
N_TC_BLK = (T_TOTAL - T_SC) // TBLK


def _pe_add_body(x_hbm, t_hbm, pe_hbm, out_hbm, *refs):
    xbs = refs[0:NSLOT]
    pe_loc = refs[NSLOT]
    idx_v = refs[NSLOT + 1]
    sin = refs[NSLOT + 2:NSLOT + 2 + NSLOT]
    sout = refs[NSLOT + 2 + NSLOT:NSLOT + 2 + 2 * NSLOT]

    wid = lax.axis_index("s") * NUM_CORES + lax.axis_index("c")
    base = wid * TS_PER_W

    pltpu.sync_copy(pe_hbm, pe_loc)
    pltpu.sync_copy(t_hbm.at[wid], idx_v.at[pl.ds(0, IDX_ROW)])

    def start_in(c, s):
        tbase = base + c * CH
        pltpu.async_copy(x_hbm.at[pl.ds(tbase, CH)], xbs[s], sin[s])

    def wait_in(c, s):
        tbase = base + c * CH
        pltpu.make_async_copy(x_hbm.at[pl.ds(tbase, CH)], xbs[s], sin[s]).wait()

    def start_out(c, s):
        tbase = base + c * CH
        pltpu.async_copy(xbs[s], out_hbm.at[pl.ds(tbase, CH)], sout[s])

    def wait_out(c, s):
        tbase = base + c * CH
        pltpu.make_async_copy(xbs[s], out_hbm.at[pl.ds(tbase, CH)], sout[s]).wait()

    def compute(c, s):
        xb = xbs[s]
        rowv = idx_v[pl.ds(c * CH, LANES)]
        for t in range(CH):
            row = rowv[t]

            def body(k, carry, t=t, row=row):
                sl = pl.ds(k * LANES, LANES)
                pv = pe_loc[row, sl]
                for b in range(B_BATCH):
                    xb[t, b, sl] = xb[t, b, sl] + pv
                return carry

            lax.fori_loop(0, DCH, body, 0, unroll=8)

    for s in range(NSLOT):
        start_in(s, s)

    def group_body(g, carry):
        c0 = g * NSLOT
        for s in range(NSLOT):
            c = c0 + s
            wait_in(c, s)
            compute(c, s)
            start_out(c, s)

            pc = c - 1
            ps = (s - 1) % NSLOT

            @pl.when(jnp.logical_and(pc >= 0, pc + NSLOT < NCHUNK))
            def _(pc=pc, ps=ps):
                wait_out(pc, ps)
                start_in(pc + NSLOT, ps)

        return carry

    lax.fori_loop(0, NGRP, group_body, 0)

    for s in range(NSLOT):
        wait_out(NCHUNK - NSLOT + s, s)


_pe_add_sc = functools.partial(
    pl.kernel,
    out_type=jax.ShapeDtypeStruct((T_SC, B_BATCH, D_MODEL), jnp.float32),
    mesh=plsc.VectorSubcoreMesh(core_axis_name="c", subcore_axis_name="s"),
    scratch_types=(
        [pltpu.VMEM((CH, B_BATCH, D_MODEL), jnp.float32) for _ in range(NSLOT)]
        + [pltpu.VMEM((PE_ROWS, D_MODEL), jnp.float32)]
        + [pltpu.VMEM((IDX_ROW + LANES,), jnp.int32)]
        + [pltpu.SemaphoreType.DMA for _ in range(2 * NSLOT)]
    ),
)(_pe_add_body)


def _tc_main_body(x_ref, idx_ref, pe_ref, o_ref):
    idxb = idx_ref[0, 0, :]
    oh = (idxb[:, None]
          == lax.broadcasted_iota(jnp.int32, (TBLK, PE_PAD), 1)).astype(
              jnp.float32)
    pos = jnp.dot(oh, pe_ref[...], preferred_element_type=jnp.float32)
    o_ref[...] = x_ref[...] + pos[:, None, :]


_tc_main = pl.pallas_call(
    _tc_main_body,
    grid=(N_TC_BLK,),
    in_specs=[
        pl.BlockSpec((TBLK, B_BATCH, D_MODEL), lambda i: (i + N_SC_BLK, 0, 0)),
        pl.BlockSpec((1, 1, TBLK), lambda i: (i + N_SC_BLK, 0, 0)),
        pl.BlockSpec((PE_PAD, D_MODEL), lambda i: (0, 0)),
    ],
    out_specs=pl.BlockSpec((TBLK, B_BATCH, D_MODEL),
                           lambda i: (i + N_SC_BLK, 0, 0)),
    out_shape=jax.ShapeDtypeStruct((T_TOTAL, B_BATCH, D_MODEL), jnp.float32),
)


def _tc_merge_body(big_ref, sc_ref, o_ref):
    o_ref[...] = sc_ref[...]


_tc_merge = pl.pallas_call(
    _tc_merge_body,
    grid=(N_SC_BLK,),
    in_specs=[
        pl.BlockSpec(memory_space=pl.ANY),
        pl.BlockSpec((TBLK, B_BATCH, D_MODEL), lambda i: (i, 0, 0)),
    ],
    out_specs=pl.BlockSpec((TBLK, B_BATCH, D_MODEL), lambda i: (i, 0, 0)),
    out_shape=jax.ShapeDtypeStruct((T_TOTAL, B_BATCH, D_MODEL), jnp.float32),
    input_output_aliases={0: 0},
)


def kernel(x, time_tensor, pe):
    idx = time_tensor.astype(jnp.int32) + OFFSET
    idx_sc = jnp.pad(idx[:T_SC].reshape(NW, TS_PER_W),
                     ((0, 0), (0, IDX_ROW - TS_PER_W)))
    idx_tc = idx.reshape(T_TOTAL // TBLK, 1, TBLK)
    pe_pad = jnp.pad(pe, ((0, PE_PAD - PE_ROWS), (0, 0)))

    sc_out = _pe_add_sc(x, idx_sc, pe)
    tc_out = _tc_main(x, idx_tc, pe_pad)
    return _tc_merge(tc_out, sc_out)
